# baseline (speedup 1.0000x reference)
import functools

import jax
import jax.numpy as jnp
import numpy as np
from jax import lax
from jax.experimental import pallas as pl
from jax.experimental.pallas import tpu as pltpu
from jax.experimental.pallas import tpu_sc as plsc

D = 128
DW = D // 2
VOCAB_BLK = 2000

N_TOK = 4096 * 200
NW = 32
PER_W = N_TOK // NW
C = 128
NCH = PER_W // C
NSET = 4


def _pack_bf16(x_lo, x_hi):
    lo = lax.bitcast_convert_type(x_lo.astype(jnp.bfloat16),
                                  jnp.uint16).astype(jnp.uint32)
    hi = lax.bitcast_convert_type(x_hi.astype(jnp.bfloat16),
                                  jnp.uint16).astype(jnp.uint32)
    return lax.bitcast_convert_type(lo | (hi << 16), jnp.int32)


def _transform_body(src_ref, wf_ref, a_lo_ref, a_hi_ref, cb_lo_ref,
                    cb_hi_ref, fpw_ref, fpb_ref, cpb_lo_ref, cpb_hi_ref,
                    srcT_ref, wfT_ref):
    a_lo = a_lo_ref[...]
    a_hi = a_hi_ref[...]
    cb_lo = cb_lo_ref[...]
    cb_hi = cb_hi_ref[...]
    fpw = fpw_ref[...]
    f32 = jnp.float32
    m_lo = jnp.dot(fpw, cb_lo, preferred_element_type=f32)
    m_hi = jnp.dot(fpw, cb_hi, preferred_element_type=f32)
    b_lo = (jnp.dot(fpb_ref[0:1, :], cb_lo, preferred_element_type=f32)
            + cpb_lo_ref[0:1, :])
    b_hi = (jnp.dot(fpb_ref[0:1, :], cb_hi, preferred_element_type=f32)
            + cpb_hi_ref[0:1, :])
    src = src_ref[...]
    wf = wf_ref[...]
    s_lo = jnp.dot(src, a_lo, preferred_element_type=f32) + b_lo
    s_hi = jnp.dot(src, a_hi, preferred_element_type=f32) + b_hi
    w_lo = jnp.dot(wf, m_lo, preferred_element_type=f32)
    w_hi = jnp.dot(wf, m_hi, preferred_element_type=f32)
    srcT_ref[...] = _pack_bf16(s_lo, s_hi)
    wfT_ref[...] = _pack_bf16(w_lo, w_hi)


def _transform_tables(src_table, wf_table, a_lo, a_hi, cb_lo, cb_hi,
                      fpw, fpb, cpb_lo, cpb_hi):
    vocab = src_table.shape[0]
    grid = (vocab // VOCAB_BLK,)
    blk = lambda r, c: pl.BlockSpec((r, c), lambda i: (0, 0))
    return pl.pallas_call(
        _transform_body,
        grid=grid,
        in_specs=[
            pl.BlockSpec((VOCAB_BLK, D), lambda i: (i, 0)),
            pl.BlockSpec((VOCAB_BLK, D), lambda i: (i, 0)),
            blk(128, DW), blk(128, DW), blk(32, DW), blk(32, DW),
            blk(128, 32), blk(8, 32), blk(8, DW), blk(8, DW),
        ],
        out_specs=[
            pl.BlockSpec((VOCAB_BLK, DW), lambda i: (i, 0)),
            pl.BlockSpec((VOCAB_BLK, DW), lambda i: (i, 0)),
        ],
        out_shape=[
            jax.ShapeDtypeStruct((vocab, DW), jnp.int32),
            jax.ShapeDtypeStruct((vocab, DW), jnp.int32),
        ],
    )(src_table, wf_table, a_lo, a_hi, cb_lo, cb_hi, fpw, fpb,
      cpb_lo, cpb_hi)


def _sc_body(tok_hbm, wfi_hbm, wn_hbm, srcT_hbm, wfT_hbm, out_hbm,
             idxA0, idxA1, idxA2, idxA3,
             idxB0, idxB1, idxB2, idxB3,
             wn0, wn1, wn2, wn3,
             bufA0, bufA1, bufA2, bufA3,
             bufB0, bufB1, bufB2, bufB3,
             bufO0, bufO1,
             semI0, semI1, semI2, semI3,
             semA0, semA1, semA2, semA3,
             semW0, semW1):
    cid = lax.axis_index("c")
    sid = lax.axis_index("s")
    wid = sid * 2 + cid
    base = wid * PER_W
    base_r = wid * (PER_W // 128)

    lane = lax.broadcasted_iota(jnp.int32, (16,), 0)
    is_last = lane == 15
    himask = jnp.full((16,), -65536, dtype=jnp.int32)

    idxA = [idxA0, idxA1, idxA2, idxA3]
    idxB = [idxB0, idxB1, idxB2, idxB3]
    wnv = [wn0, wn1, wn2, wn3]
    bufA = [bufA0, bufA1, bufA2, bufA3]
    bufB = [bufB0, bufB1, bufB2, bufB3]
    bufO = [bufO0, bufO1]
    semI = [semI0, semI1, semI2, semI3]
    semA = [semA0, semA1, semA2, semA3]
    semW = [semW0, semW1]

    def out_slice(c):
        return out_hbm.at[pl.ds(base + c * C, C)]

    def fire_idx(s, c):
        r0 = base_r + c
        pltpu.async_copy(tok_hbm.at[pl.ds(r0, 1)], idxA[s], semI[s])
        pltpu.async_copy(wfi_hbm.at[pl.ds(r0, 1)], idxB[s], semI[s])
        pltpu.async_copy(wn_hbm.at[pl.ds(r0, 1)], wnv[s], semI[s])

    def wait_idx(s, c):
        r0 = base_r + c
        pltpu.make_async_copy(tok_hbm.at[pl.ds(r0, 1)], idxA[s],
                              semI[s]).wait()
        pltpu.make_async_copy(wfi_hbm.at[pl.ds(r0, 1)], idxB[s],
                              semI[s]).wait()
        pltpu.make_async_copy(wn_hbm.at[pl.ds(r0, 1)], wnv[s],
                              semI[s]).wait()

    def fire_g(s, c):
        wait_idx(s, c)
        pltpu.async_copy(srcT_hbm.at[idxA[s].at[0]], bufA[s], semA[s])
        pltpu.async_copy(wfT_hbm.at[idxB[s].at[0]], bufB[s], semA[s])

    def finish(s, o, c):
        pltpu.make_async_copy(srcT_hbm.at[idxA[s].at[0]], bufA[s],
                              semA[s]).wait()
        pltpu.make_async_copy(wfT_hbm.at[idxB[s].at[0]], bufB[s],
                              semA[s]).wait()

        @pl.when(c >= 2)
        def _():
            pltpu.make_async_copy(bufO[o], out_slice(c), semW[o]).wait()

        def grp_body(g, c2):
            wn_grp = wnv[s][0, pl.ds(g * 16, 16)]
            for j in range(16):
                r = g * 16 + j
                for q in range(4):
                    sl = pl.ds(q * 16, 16)
                    wa = bufA[s][r, sl]
                    wb = bufB[s][r, sl]
                    lo = (lax.bitcast_convert_type(wa << 16, jnp.float32)
                          + lax.bitcast_convert_type(wb << 16, jnp.float32))
                    hi = (lax.bitcast_convert_type(wa & himask, jnp.float32)
                          + lax.bitcast_convert_type(wb & himask, jnp.float32))
                    if q == 3:
                        hi = jnp.where(is_last, wn_grp[j], hi)
                    bufO[o][r, pl.ds(q * 32, 16)] = lo
                    bufO[o][r, pl.ds(q * 32 + 16, 16)] = hi
            return c2
        lax.fori_loop(0, C // 16, grp_body, 0)
        pltpu.async_copy(bufO[o], out_slice(c), semW[o])

    fire_idx(0, 0)
    fire_idx(1, 1)
    fire_idx(2, 2)
    fire_g(0, 0)
    fire_g(1, 1)

    def body4(t, carry):
        for u in range(NSET):
            i = NSET * t + u

            @pl.when(i + 3 < NCH)
            def _():
                fire_idx((u + 3) % NSET, i + 3)

            @pl.when(i + 2 < NCH)
            def _():
                fire_g((u + 2) % NSET, i + 2)

            finish(u, u % 2, i)
        return carry

    lax.fori_loop(0, NCH // NSET, body4, 0)

    for u in range(2):
        pltpu.make_async_copy(bufO[u], out_slice(NCH - 2 + u),
                              semW[u]).wait()


def _sc_gather(tok, wfi, wn, srcT, wfT):
    mesh = plsc.VectorSubcoreMesh(core_axis_name="c", subcore_axis_name="s")
    f = functools.partial(
        pl.kernel, _sc_body, mesh=mesh,
        compiler_params=pltpu.CompilerParams(use_tc_tiling_on_sc=False),
        out_type=jax.ShapeDtypeStruct((N_TOK, D), jnp.float32),
        scratch_types=(
            [pltpu.VMEM((1, 128), jnp.int32)] * 8
            + [pltpu.VMEM((1, 128), jnp.float32)] * 4
            + [pltpu.VMEM((C, DW), jnp.int32)] * 8
            + [pltpu.VMEM((C, D), jnp.float32)] * 2
            + [pltpu.SemaphoreType.DMA] * 10
        ),
    )()
    return f(tok, wfi, wn, srcT, wfT)


def kernel(token, weighted_factor, weighted_notes, src_table, wf_table,
           fp_W, fp_b, cp_W, cp_b):
    tok = token.astype(jnp.int32).reshape(N_TOK // 128, 128)
    wfi = weighted_factor.astype(jnp.int32).reshape(N_TOK // 128, 128)
    wn = weighted_notes.astype(jnp.float32).reshape(N_TOK // 128, 128)

    perm_lo = np.concatenate([np.arange(32 * g, 32 * g + 16)
                              for g in range(4)])
    perm_hi = np.concatenate([np.arange(32 * g + 16, 32 * g + 32)
                              for g in range(4)])

    cp_top = jnp.zeros((128, 128), jnp.float32).at[:, :127].set(cp_W[:128])
    cp_bot = jnp.zeros((32, 128), jnp.float32).at[:25, :127].set(cp_W[128:])
    cpb = jnp.zeros((8, 128), jnp.float32).at[0, :127].set(cp_b)
    a_lo = cp_top[:, perm_lo]
    a_hi = cp_top[:, perm_hi]
    cb_lo = cp_bot[:, perm_lo]
    cb_hi = cp_bot[:, perm_hi]
    cpb_lo = cpb[:, perm_lo]
    cpb_hi = cpb[:, perm_hi]
    fpw = jnp.zeros((128, 32), jnp.float32).at[:, :25].set(fp_W)
    fpb = jnp.zeros((8, 32), jnp.float32).at[0, :25].set(fp_b)

    srcT, wfT = _transform_tables(src_table, wf_table, a_lo, a_hi,
                                  cb_lo, cb_hi, fpw, fpb, cpb_lo, cpb_hi)
    out = _sc_gather(tok, wfi, wn, srcT, wfT)
    return out.reshape(4096, 200, D)


# speedup vs baseline: 1.6045x; 1.6045x over previous
import functools

import jax
import jax.numpy as jnp
from jax import lax
from jax.experimental import pallas as pl
from jax.experimental.pallas import tpu as pltpu
from jax.experimental.pallas import tpu_sc as plsc

D = 128
VOCAB_BLK = 4000

N_TOK = 4096 * 200
NW = 32
PER_W = N_TOK // NW
C = 128
NCH = PER_W // C


def _transform_body(src_ref, wf_ref, cp_top_ref, cp_bot_ref, fpw_ref,
                    fpb_ref, cpb_ref, srcT_ref, wfT_ref):
    cp_top = cp_top_ref[...]
    cp_bot = cp_bot_ref[...]
    m = jnp.dot(fpw_ref[...], cp_bot, preferred_element_type=jnp.float32)
    bias = (jnp.dot(fpb_ref[0:1, :], cp_bot,
                    preferred_element_type=jnp.float32) + cpb_ref[0:1, :])
    srcT_ref[...] = jnp.dot(src_ref[...], cp_top,
                            preferred_element_type=jnp.float32) + bias
    wfT_ref[...] = jnp.dot(wf_ref[...], m,
                           preferred_element_type=jnp.float32)


def _transform_tables(src_table, wf_table, cp_top, cp_bot, fpw, fpb, cpb):
    vocab = src_table.shape[0]
    grid = (vocab // VOCAB_BLK,)
    blk = lambda r, c: pl.BlockSpec((r, c), lambda i: (0, 0))
    return pl.pallas_call(
        _transform_body,
        grid=grid,
        in_specs=[
            pl.BlockSpec((VOCAB_BLK, D), lambda i: (i, 0)),
            pl.BlockSpec((VOCAB_BLK, D), lambda i: (i, 0)),
            blk(128, 128), blk(32, 128), blk(128, 32), blk(8, 32),
            blk(8, 128),
        ],
        out_specs=[
            pl.BlockSpec((VOCAB_BLK, D), lambda i: (i, 0)),
            pl.BlockSpec((VOCAB_BLK, D), lambda i: (i, 0)),
        ],
        out_shape=[
            jax.ShapeDtypeStruct((vocab, D), jnp.float32),
            jax.ShapeDtypeStruct((vocab, D), jnp.float32),
        ],
    )(src_table, wf_table, cp_top, cp_bot, fpw, fpb, cpb)


NSET = 5


def _sc_body(tok_hbm, wfi_hbm, wn_hbm, srcT_hbm, wfT_hbm, out_hbm,
             idxA0, idxA1, idxA2, idxA3, idxA4,
             idxB0, idxB1, idxB2, idxB3, idxB4,
             wn0, wn1, wn2, wn3, wn4,
             bufG0, bufG1, bufG2, bufG3, bufG4,
             semI0, semI1, semI2, semI3, semI4,
             semA0, semA1, semA2, semA3, semA4,
             semB0, semB1, semB2, semB3, semB4,
             semW0, semW1, semW2, semW3, semW4):
    cid = lax.axis_index("c")
    sid = lax.axis_index("s")
    wid = sid * 2 + cid
    base = wid * PER_W
    base_r = wid * (PER_W // 128)

    lane = lax.broadcasted_iota(jnp.int32, (16,), 0)
    is_last = lane == 15

    idxA = [idxA0, idxA1, idxA2, idxA3, idxA4]
    idxB = [idxB0, idxB1, idxB2, idxB3, idxB4]
    wnv = [wn0, wn1, wn2, wn3, wn4]
    bufG = [bufG0, bufG1, bufG2, bufG3, bufG4]
    semI = [semI0, semI1, semI2, semI3, semI4]
    semA = [semA0, semA1, semA2, semA3, semA4]
    semB = [semB0, semB1, semB2, semB3, semB4]
    semW = [semW0, semW1, semW2, semW3, semW4]

    def out_slice(c):
        return out_hbm.at[pl.ds(base + c * C, C)]

    def fire_idx(s, c):
        r0 = base_r + c
        pltpu.async_copy(tok_hbm.at[pl.ds(r0, 1)], idxA[s], semI[s])
        pltpu.async_copy(wfi_hbm.at[pl.ds(r0, 1)], idxB[s], semI[s])
        pltpu.async_copy(wn_hbm.at[pl.ds(r0, 1)], wnv[s], semI[s])

    def wait_idx(s, c):
        r0 = base_r + c
        pltpu.make_async_copy(tok_hbm.at[pl.ds(r0, 1)], idxA[s],
                              semI[s]).wait()
        pltpu.make_async_copy(wfi_hbm.at[pl.ds(r0, 1)], idxB[s],
                              semI[s]).wait()
        pltpu.make_async_copy(wn_hbm.at[pl.ds(r0, 1)], wnv[s],
                              semI[s]).wait()

    def fire_a(s, c, first):
        if not first:
            @pl.when(c >= NSET)
            def _():
                pltpu.make_async_copy(bufG[s], out_slice(c), semW[s]).wait()
        wait_idx(s, c)
        pltpu.async_copy(srcT_hbm.at[idxA[s].at[0]], bufG[s], semA[s])

    def fire_b(s):
        pltpu.make_async_copy(srcT_hbm.at[idxA[s].at[0]], bufG[s],
                              semA[s]).wait()
        pltpu.async_copy(wfT_hbm.at[idxB[s].at[0]], bufG[s], semB[s],
                         add=True)

    def finish(s, c):
        pltpu.make_async_copy(wfT_hbm.at[idxB[s].at[0]], bufG[s],
                              semB[s]).wait()

        def grp_body(g, c2):
            wn_grp = wnv[s][0, pl.ds(g * 16, 16)]
            for j in range(16):
                r = g * 16 + j
                sl = pl.ds(D - 16, 16)
                bufG[s][r, sl] = jnp.where(is_last, wn_grp[j],
                                           bufG[s][r, sl])
            return c2
        lax.fori_loop(0, C // 16, grp_body, 0)
        pltpu.async_copy(bufG[s], out_slice(c), semW[s])

    fire_idx(0, 0)
    fire_idx(1, 1)
    fire_idx(2, 2)
    fire_idx(3, 3)
    fire_a(0, 0, True)
    fire_b(0)
    fire_a(1, 1, True)
    fire_a(2, 2, True)

    def body4(t, carry):
        for u in range(NSET):
            i = NSET * t + u

            @pl.when(i + 4 < NCH)
            def _():
                fire_idx((u + 4) % NSET, i + 4)

            @pl.when(i + 3 < NCH)
            def _():
                fire_a((u + 3) % NSET, i + 3, False)

            @pl.when(i + 1 < NCH)
            def _():
                fire_b((u + 1) % NSET)

            finish(u, i)
        return carry

    lax.fori_loop(0, NCH // NSET, body4, 0)

    for u in range(NSET):
        pltpu.make_async_copy(bufG[u], out_slice(NCH - NSET + u),
                              semW[u]).wait()


def _sc_gather(tok, wfi, wn, srcT, wfT):
    mesh = plsc.VectorSubcoreMesh(core_axis_name="c", subcore_axis_name="s")
    f = functools.partial(
        pl.kernel, _sc_body, mesh=mesh,
        out_type=jax.ShapeDtypeStruct((N_TOK, D), jnp.float32),
        scratch_types=(
            [pltpu.VMEM((1, 128), jnp.int32)] * 10
            + [pltpu.VMEM((1, 128), jnp.float32)] * 5
            + [pltpu.VMEM((C, D), jnp.float32)] * 5
            + [pltpu.SemaphoreType.DMA] * 20
        ),
    )()
    return f(tok, wfi, wn, srcT, wfT)


def kernel(token, weighted_factor, weighted_notes, src_table, wf_table,
           fp_W, fp_b, cp_W, cp_b):
    tok = token.astype(jnp.int32).reshape(N_TOK // 128, 128)
    wfi = weighted_factor.astype(jnp.int32).reshape(N_TOK // 128, 128)
    wn = weighted_notes.astype(jnp.float32).reshape(N_TOK // 128, 128)

    cp_top = jnp.zeros((128, 128), jnp.float32).at[:, :127].set(cp_W[:128])
    cp_bot = jnp.zeros((32, 128), jnp.float32).at[:25, :127].set(cp_W[128:])
    fpw = jnp.zeros((128, 32), jnp.float32).at[:, :25].set(fp_W)
    fpb = jnp.zeros((8, 32), jnp.float32).at[0, :25].set(fp_b)
    cpb = jnp.zeros((8, 128), jnp.float32).at[0, :127].set(cp_b)

    srcT, wfT = _transform_tables(src_table, wf_table, cp_top, cp_bot,
                                  fpw, fpb, cpb)
    out = _sc_gather(tok, wfi, wn, srcT, wfT)
    return out.reshape(4096, 200, D)


# speedup vs baseline: 1.6108x; 1.0039x over previous
import functools

import jax
import jax.numpy as jnp
from jax import lax
from jax.experimental import pallas as pl
from jax.experimental.pallas import tpu as pltpu
from jax.experimental.pallas import tpu_sc as plsc

D = 128
VOCAB_BLK = 10000

N_TOK = 4096 * 200
NW = 32
PER_W = N_TOK // NW
C = 128
NCH = PER_W // C


def _transform_body(src_ref, wf_ref, cp_top_ref, cp_bot_ref, fpw_ref,
                    fpb_ref, cpb_ref, srcT_ref, wfT_ref):
    cp_top = cp_top_ref[...]
    cp_bot = cp_bot_ref[...]
    m = jnp.dot(fpw_ref[...], cp_bot, preferred_element_type=jnp.float32)
    bias = (jnp.dot(fpb_ref[0:1, :], cp_bot,
                    preferred_element_type=jnp.float32) + cpb_ref[0:1, :])
    srcT_ref[...] = jnp.dot(src_ref[...], cp_top,
                            preferred_element_type=jnp.float32) + bias
    wfT_ref[...] = jnp.dot(wf_ref[...], m,
                           preferred_element_type=jnp.float32)


def _transform_tables(src_table, wf_table, cp_top, cp_bot, fpw, fpb, cpb):
    vocab = src_table.shape[0]
    grid = (vocab // VOCAB_BLK,)
    blk = lambda r, c: pl.BlockSpec((r, c), lambda i: (0, 0))
    return pl.pallas_call(
        _transform_body,
        grid=grid,
        in_specs=[
            pl.BlockSpec((VOCAB_BLK, D), lambda i: (i, 0)),
            pl.BlockSpec((VOCAB_BLK, D), lambda i: (i, 0)),
            blk(128, 128), blk(32, 128), blk(128, 32), blk(8, 32),
            blk(8, 128),
        ],
        out_specs=[
            pl.BlockSpec((VOCAB_BLK, D), lambda i: (i, 0)),
            pl.BlockSpec((VOCAB_BLK, D), lambda i: (i, 0)),
        ],
        out_shape=[
            jax.ShapeDtypeStruct((vocab, D), jnp.float32),
            jax.ShapeDtypeStruct((vocab, D), jnp.float32),
        ],
    )(src_table, wf_table, cp_top, cp_bot, fpw, fpb, cpb)


NSET = 5


def _sc_body(tok_hbm, wfi_hbm, wn_hbm, srcT_hbm, wfT_hbm, out_hbm,
             idxA0, idxA1, idxA2, idxA3, idxA4,
             idxB0, idxB1, idxB2, idxB3, idxB4,
             wn0, wn1, wn2, wn3, wn4,
             bufG0, bufG1, bufG2, bufG3, bufG4,
             semI0, semI1, semI2, semI3, semI4,
             semA0, semA1, semA2, semA3, semA4,
             semB0, semB1, semB2, semB3, semB4,
             semW0, semW1, semW2, semW3, semW4):
    cid = lax.axis_index("c")
    sid = lax.axis_index("s")
    wid = sid * 2 + cid
    base = wid * PER_W
    base_r = wid * (PER_W // 128)

    lane = lax.broadcasted_iota(jnp.int32, (16,), 0)
    is_last = lane == 15

    idxA = [idxA0, idxA1, idxA2, idxA3, idxA4]
    idxB = [idxB0, idxB1, idxB2, idxB3, idxB4]
    wnv = [wn0, wn1, wn2, wn3, wn4]
    bufG = [bufG0, bufG1, bufG2, bufG3, bufG4]
    semI = [semI0, semI1, semI2, semI3, semI4]
    semA = [semA0, semA1, semA2, semA3, semA4]
    semB = [semB0, semB1, semB2, semB3, semB4]
    semW = [semW0, semW1, semW2, semW3, semW4]

    def out_slice(c):
        return out_hbm.at[pl.ds(base + c * C, C)]

    def fire_idx(s, c):
        r0 = base_r + c
        pltpu.async_copy(tok_hbm.at[pl.ds(r0, 1)], idxA[s], semI[s])
        pltpu.async_copy(wfi_hbm.at[pl.ds(r0, 1)], idxB[s], semI[s])
        pltpu.async_copy(wn_hbm.at[pl.ds(r0, 1)], wnv[s], semI[s])

    def wait_idx(s, c):
        r0 = base_r + c
        pltpu.make_async_copy(tok_hbm.at[pl.ds(r0, 1)], idxA[s],
                              semI[s]).wait()
        pltpu.make_async_copy(wfi_hbm.at[pl.ds(r0, 1)], idxB[s],
                              semI[s]).wait()
        pltpu.make_async_copy(wn_hbm.at[pl.ds(r0, 1)], wnv[s],
                              semI[s]).wait()

    def fire_a(s, c, first):
        if not first:
            @pl.when(c >= NSET)
            def _():
                pltpu.make_async_copy(bufG[s], out_slice(c), semW[s]).wait()
        wait_idx(s, c)
        pltpu.async_copy(srcT_hbm.at[idxA[s].at[0]], bufG[s], semA[s])

    def fire_b(s):
        pltpu.make_async_copy(srcT_hbm.at[idxA[s].at[0]], bufG[s],
                              semA[s]).wait()
        pltpu.async_copy(wfT_hbm.at[idxB[s].at[0]], bufG[s], semB[s],
                         add=True)

    def finish(s, c):
        pltpu.make_async_copy(wfT_hbm.at[idxB[s].at[0]], bufG[s],
                              semB[s]).wait()

        def grp_body(g, c2):
            wn_grp = wnv[s][0, pl.ds(g * 16, 16)]
            for j in range(16):
                r = g * 16 + j
                sl = pl.ds(D - 16, 16)
                bufG[s][r, sl] = jnp.where(is_last, wn_grp[j],
                                           bufG[s][r, sl])
            return c2
        lax.fori_loop(0, C // 16, grp_body, 0)
        pltpu.async_copy(bufG[s], out_slice(c), semW[s])

    fire_idx(0, 0)
    fire_idx(1, 1)
    fire_idx(2, 2)
    fire_idx(3, 3)
    fire_a(0, 0, True)
    fire_b(0)
    fire_a(1, 1, True)
    fire_a(2, 2, True)

    def body4(t, carry):
        for u in range(NSET):
            i = NSET * t + u

            @pl.when(i + 4 < NCH)
            def _():
                fire_idx((u + 4) % NSET, i + 4)

            @pl.when(i + 3 < NCH)
            def _():
                fire_a((u + 3) % NSET, i + 3, False)

            @pl.when(i + 1 < NCH)
            def _():
                fire_b((u + 1) % NSET)

            finish(u, i)
        return carry

    lax.fori_loop(0, NCH // NSET, body4, 0)

    for u in range(NSET):
        pltpu.make_async_copy(bufG[u], out_slice(NCH - NSET + u),
                              semW[u]).wait()


def _sc_gather(tok, wfi, wn, srcT, wfT):
    mesh = plsc.VectorSubcoreMesh(core_axis_name="c", subcore_axis_name="s")
    f = functools.partial(
        pl.kernel, _sc_body, mesh=mesh,
        out_type=jax.ShapeDtypeStruct((N_TOK, D), jnp.float32),
        scratch_types=(
            [pltpu.VMEM((1, 128), jnp.int32)] * 10
            + [pltpu.VMEM((1, 128), jnp.float32)] * 5
            + [pltpu.VMEM((C, D), jnp.float32)] * 5
            + [pltpu.SemaphoreType.DMA] * 20
        ),
    )()
    return f(tok, wfi, wn, srcT, wfT)


def kernel(token, weighted_factor, weighted_notes, src_table, wf_table,
           fp_W, fp_b, cp_W, cp_b):
    tok = token.astype(jnp.int32).reshape(N_TOK // 128, 128)
    wfi = weighted_factor.astype(jnp.int32).reshape(N_TOK // 128, 128)
    wn = weighted_notes.astype(jnp.float32).reshape(N_TOK // 128, 128)

    cp_top = jnp.zeros((128, 128), jnp.float32).at[:, :127].set(cp_W[:128])
    cp_bot = jnp.zeros((32, 128), jnp.float32).at[:25, :127].set(cp_W[128:])
    fpw = jnp.zeros((128, 32), jnp.float32).at[:, :25].set(fp_W)
    fpb = jnp.zeros((8, 32), jnp.float32).at[0, :25].set(fp_b)
    cpb = jnp.zeros((8, 128), jnp.float32).at[0, :127].set(cp_b)

    srcT, wfT = _transform_tables(src_table, wf_table, cp_top, cp_bot,
                                  fpw, fpb, cpb)
    out = _sc_gather(tok, wfi, wn, srcT, wfT)
    return out.reshape(4096, 200, D)
